# baseline (speedup 1.0000x reference)
import functools

import jax
import jax.numpy as jnp
from jax import lax
from jax.experimental import pallas as pl
from jax.experimental.pallas import tpu as pltpu
from jax.experimental.pallas import tpu_sc as plsc


def _order_map_sc(B, N, D, I):
    info = plsc.get_sparse_core_info()
    NC, NS, L = info.num_cores, info.num_subcores, info.num_lanes
    NW = NC * NS
    total = B * I
    per_w = total // NW
    assert total % NW == 0 and per_w % L == 0 and I % per_w == 0

    mesh = plsc.VectorSubcoreMesh(core_axis_name="c", subcore_axis_name="s")

    @functools.partial(
        pl.kernel,
        mesh=mesh,
        out_type=jax.ShapeDtypeStruct((total, D), jnp.float32),
        scratch_types=[
            pltpu.VMEM((per_w,), jnp.int32),
            pltpu.VMEM((per_w, D), jnp.float32),
            pltpu.SemaphoreType.DMA,
        ],
        compiler_params=pltpu.CompilerParams(use_tc_tiling_on_sc=False),
    )
    def gather_kernel(x_hbm, idx_hbm, out_hbm, idx_v, rows_v, sem):
        wid = lax.axis_index("s") * NC + lax.axis_index("c")
        row0 = wid * per_w
        b = row0 // I
        i0 = row0 - b * I
        pltpu.sync_copy(idx_hbm.at[pl.ds(i0, per_w)], idx_v)
        base = b * N
        for j in range(per_w // L):
            v = idx_v[pl.ds(j * L, L)]
            v = jnp.minimum(jnp.maximum(v, 0), N - 1) + base
            idx_v[pl.ds(j * L, L)] = v
        pltpu.async_copy(x_hbm.at[idx_v], rows_v, sem).wait()
        pltpu.sync_copy(rows_v, out_hbm.at[pl.ds(row0, per_w)])

    return gather_kernel


def kernel(x, indices):
    B, N, D = x.shape
    I = indices.shape[0]
    xf = x.astype(jnp.float32)
    out = _order_map_sc(B, N, D, I)(xf.reshape(B * N, D), indices)
    return out.reshape(B, I, D)
